# baseline (speedup 1.0000x reference)
import dataclasses
import functools

import jax
import jax.numpy as jnp
from jax import lax
from jax.experimental import pallas as pl
from jax.experimental.pallas import tpu as pltpu
from jax.experimental.pallas import tpu_sc as plsc

N = 100000
D = 128
H = 64
S = 512

NC = 2
NS = 16
L = 16
NW = NC * NS

P = 2
NPAD = 102400
NPP = NPAD // P
CHUNK = NPP // NW

BLK = 10240
GPP = NPP // BLK


def _mlp_body(x_ref, w1h_ref, w1l_ref, b1_ref, w2_ref, b2_ref, o_ref, *, phase):
    x = x_ref[...]
    xh = x.astype(jnp.bfloat16)
    xl = (x - xh.astype(jnp.float32)).astype(jnp.bfloat16)
    wh = w1h_ref[...]
    wl = w1l_ref[...]
    h = jnp.dot(xh, wh, preferred_element_type=jnp.float32)
    h = h + (jnp.dot(xh, wl, preferred_element_type=jnp.float32)
             + jnp.dot(xl, wh, preferred_element_type=jnp.float32))
    h = h + b1_ref[...]
    h = h * jax.nn.sigmoid(h)
    r = jnp.sum(h * w2_ref[...], axis=1) + b2_ref[0, 0]
    gidx = (phase * NPP + pl.program_id(0) * BLK) + lax.iota(jnp.int32, BLK)
    o_ref[...] = jnp.where(gidx < N, r, 0.0)


def _mlp(x, w1h, w1l, b1r, w2r, b2r, phase):
    return pl.pallas_call(
        functools.partial(_mlp_body, phase=phase),
        grid=(GPP,),
        in_specs=[
            pl.BlockSpec((BLK, D), lambda i: (i + phase * GPP, 0)),
            pl.BlockSpec((D, H), lambda i: (0, 0)),
            pl.BlockSpec((D, H), lambda i: (0, 0)),
            pl.BlockSpec((1, H), lambda i: (0, 0)),
            pl.BlockSpec((1, H), lambda i: (0, 0)),
            pl.BlockSpec((1, 1), lambda i: (0, 0)),
        ],
        out_specs=pl.BlockSpec((BLK,), lambda i: (i,)),
        out_shape=jax.ShapeDtypeStruct((NPP,), jnp.float32),
        compiler_params=pltpu.CompilerParams(
            dimension_semantics=("arbitrary",)),
    )(x, w1h, w1l, b1r, w2r, b2r)


def _segsum(res_phase, idx_pad, phase):
    mesh = plsc.VectorSubcoreMesh(core_axis_name="c", subcore_axis_name="s")
    cp = pltpu.CompilerParams()
    if "needs_layout_passes" in pltpu.CompilerParams.__dataclass_fields__:
        cp = dataclasses.replace(cp, needs_layout_passes=False)

    @functools.partial(
        pl.kernel,
        compiler_params=cp,
        out_type=jax.ShapeDtypeStruct((NC, S), jnp.float32),
        mesh=mesh,
        scratch_types=[
            pltpu.VMEM((CHUNK,), jnp.float32),
            pltpu.VMEM((CHUNK,), jnp.int32),
            pltpu.VMEM((S,), jnp.float32),
            pltpu.VMEM((S,), jnp.int32),
            pltpu.VMEM_SHARED((S,), jnp.float32),
        ],
    )
    def k(res_hbm, idx_hbm, out_hbm, res_v, idx_v, acc_v, iota_v, shared):
        c = lax.axis_index("c")
        s = lax.axis_index("s")
        wid = c * NS + s
        pltpu.sync_copy(res_hbm.at[pl.ds(wid * CHUNK, CHUNK)], res_v)
        pltpu.sync_copy(
            idx_hbm.at[pl.ds(phase * NPP + wid * CHUNK, CHUNK)], idx_v)

        @pl.loop(0, S, step=L)
        def _zero(i):
            acc_v[pl.ds(i, L)] = jnp.zeros((L,), jnp.float32)
            iota_v[pl.ds(i, L)] = lax.iota(jnp.int32, L) + i

        @pl.when(s == 0)
        def _init_shared():
            pltpu.sync_copy(acc_v, shared)

        @pl.loop(0, CHUNK, step=L, unroll=8)
        def _scatter(i):
            plsc.addupdate_scatter(
                acc_v, [idx_v[pl.ds(i, L)]], res_v[pl.ds(i, L)])

        plsc.subcore_barrier()
        pltpu.sync_copy(acc_v, shared.at[iota_v], add=True)
        plsc.subcore_barrier()

        @pl.when(s == 0)
        def _writeout():
            pltpu.sync_copy(shared, out_hbm.at[c])

    return k(res_phase, idx_pad)


def kernel(node_scalar, batch, W1, b1, W2, b2):
    b1r = b1.reshape(1, H)
    w2r = W2.reshape(1, H)
    b2r = b2.reshape(1, 1)
    w1h = W1.astype(jnp.bfloat16)
    w1l = (W1 - w1h.astype(jnp.float32)).astype(jnp.bfloat16)
    idx_flat = jnp.concatenate(
        [batch.astype(jnp.int32), jnp.zeros((NPAD - N,), jnp.int32)])
    out = jnp.zeros((S,), jnp.float32)
    for p in range(P):
        res_p = _mlp(node_scalar, w1h, w1l, b1r, w2r, b2r, p)
        partials = _segsum(res_p, idx_flat, p)
        out = out + partials[0] + partials[1]
    return out


# speedup vs baseline: 1.1182x; 1.1182x over previous
import dataclasses
import functools

import jax
import jax.numpy as jnp
from jax import lax
from jax.experimental import pallas as pl
from jax.experimental.pallas import tpu as pltpu
from jax.experimental.pallas import tpu_sc as plsc

N = 100000
D = 128
H = 64
S = 512

NC = 2
NS = 16
L = 16
NW = NC * NS

P = 2
NPAD = 102400
NPP = NPAD // P
CHUNK = NPP // NW

BLK = 5120
GPP = NPP // BLK


def _mlp_body(x_ref, w1_ref, b1_ref, w2_ref, b2_ref, o_ref, *, phase):
    x = x_ref[...]
    h = jnp.dot(x, w1_ref[...], preferred_element_type=jnp.float32)
    h = h + b1_ref[...]
    h = h * jax.nn.sigmoid(h)
    r = jnp.sum(h * w2_ref[...], axis=1) + b2_ref[0, 0]
    gidx = (phase * NPP + pl.program_id(0) * BLK) + lax.iota(jnp.int32, BLK)
    o_ref[...] = jnp.where(gidx < N, r, 0.0)


def _mlp(x, w1, b1r, w2r, b2r, phase):
    return pl.pallas_call(
        functools.partial(_mlp_body, phase=phase),
        grid=(GPP,),
        in_specs=[
            pl.BlockSpec((BLK, D), lambda i: (i + phase * GPP, 0)),
            pl.BlockSpec((D, H), lambda i: (0, 0)),
            pl.BlockSpec((1, H), lambda i: (0, 0)),
            pl.BlockSpec((1, H), lambda i: (0, 0)),
            pl.BlockSpec((1, 1), lambda i: (0, 0)),
        ],
        out_specs=pl.BlockSpec((BLK,), lambda i: (i,)),
        out_shape=jax.ShapeDtypeStruct((NPP,), jnp.float32),
        compiler_params=pltpu.CompilerParams(
            dimension_semantics=("arbitrary",)),
    )(x, w1, b1r, w2r, b2r)


def _segsum(res_phase, idx_pad, phase):
    mesh = plsc.VectorSubcoreMesh(core_axis_name="c", subcore_axis_name="s")
    cp = pltpu.CompilerParams()
    if "needs_layout_passes" in pltpu.CompilerParams.__dataclass_fields__:
        cp = dataclasses.replace(cp, needs_layout_passes=False)

    @functools.partial(
        pl.kernel,
        compiler_params=cp,
        out_type=jax.ShapeDtypeStruct((NC, S), jnp.float32),
        mesh=mesh,
        scratch_types=[
            pltpu.VMEM((CHUNK,), jnp.float32),
            pltpu.VMEM((CHUNK,), jnp.int32),
            pltpu.VMEM((S,), jnp.float32),
            pltpu.VMEM((S,), jnp.int32),
            pltpu.VMEM_SHARED((S,), jnp.float32),
        ],
    )
    def k(res_hbm, idx_hbm, out_hbm, res_v, idx_v, acc_v, iota_v, shared):
        c = lax.axis_index("c")
        s = lax.axis_index("s")
        wid = c * NS + s
        pltpu.sync_copy(res_hbm.at[pl.ds(wid * CHUNK, CHUNK)], res_v)
        pltpu.sync_copy(
            idx_hbm.at[pl.ds(phase * NPP + wid * CHUNK, CHUNK)], idx_v)

        @pl.loop(0, S, step=L)
        def _zero(i):
            acc_v[pl.ds(i, L)] = jnp.zeros((L,), jnp.float32)
            iota_v[pl.ds(i, L)] = lax.iota(jnp.int32, L) + i

        @pl.when(s == 0)
        def _init_shared():
            pltpu.sync_copy(acc_v, shared)

        @pl.loop(0, CHUNK, step=L, unroll=8)
        def _scatter(i):
            plsc.addupdate_scatter(
                acc_v, [idx_v[pl.ds(i, L)]], res_v[pl.ds(i, L)])

        plsc.subcore_barrier()
        pltpu.sync_copy(acc_v, shared.at[iota_v], add=True)
        plsc.subcore_barrier()

        @pl.when(s == 0)
        def _writeout():
            pltpu.sync_copy(shared, out_hbm.at[c])

    return k(res_phase, idx_pad)


def kernel(node_scalar, batch, W1, b1, W2, b2):
    b1r = b1.reshape(1, H)
    w2r = W2.reshape(1, H)
    b2r = b2.reshape(1, 1)
    idx_flat = jnp.concatenate(
        [batch.astype(jnp.int32), jnp.zeros((NPAD - N,), jnp.int32)])
    out = jnp.zeros((S,), jnp.float32)
    for p in range(P):
        res_p = _mlp(node_scalar, W1, b1r, w2r, b2r, p)
        partials = _segsum(res_p, idx_flat, p)
        out = out + partials[0] + partials[1]
    return out


# speedup vs baseline: 1.1607x; 1.0380x over previous
import dataclasses
import functools

import jax
import jax.numpy as jnp
from jax import lax
from jax.experimental import pallas as pl
from jax.experimental.pallas import tpu as pltpu
from jax.experimental.pallas import tpu_sc as plsc

N = 100000
D = 128
H = 64
S = 512

NC = 2
NS = 16
L = 16
NW = NC * NS

P = 2
NPAD = 102400
NPP = NPAD // P
CHUNK = NPP // NW

BLK = 25600
GPP = NPP // BLK


def _mlp_body(x_ref, w1_ref, b1_ref, w2_ref, b2_ref, o_ref, *, phase):
    x = x_ref[...]
    h = jnp.dot(x, w1_ref[...], preferred_element_type=jnp.float32)
    h = h + b1_ref[...]
    h = h * jax.nn.sigmoid(h)
    r = jnp.sum(h * w2_ref[...], axis=1) + b2_ref[0, 0]
    gidx = (phase * NPP + pl.program_id(0) * BLK) + lax.iota(jnp.int32, BLK)
    o_ref[...] = jnp.where(gidx < N, r, 0.0)


def _mlp(x, w1, b1r, w2r, b2r, phase):
    return pl.pallas_call(
        functools.partial(_mlp_body, phase=phase),
        grid=(GPP,),
        in_specs=[
            pl.BlockSpec((BLK, D), lambda i: (i + phase * GPP, 0)),
            pl.BlockSpec((D, H), lambda i: (0, 0)),
            pl.BlockSpec((1, H), lambda i: (0, 0)),
            pl.BlockSpec((1, H), lambda i: (0, 0)),
            pl.BlockSpec((1, 1), lambda i: (0, 0)),
        ],
        out_specs=pl.BlockSpec((BLK,), lambda i: (i,)),
        out_shape=jax.ShapeDtypeStruct((NPP,), jnp.float32),
        compiler_params=pltpu.CompilerParams(
            dimension_semantics=("arbitrary",)),
    )(x, w1, b1r, w2r, b2r)


def _segsum(res_phase, idx_pad, phase):
    mesh = plsc.VectorSubcoreMesh(core_axis_name="c", subcore_axis_name="s")
    cp = pltpu.CompilerParams()
    if "needs_layout_passes" in pltpu.CompilerParams.__dataclass_fields__:
        cp = dataclasses.replace(cp, needs_layout_passes=False)

    @functools.partial(
        pl.kernel,
        compiler_params=cp,
        out_type=jax.ShapeDtypeStruct((NC, S), jnp.float32),
        mesh=mesh,
        scratch_types=[
            pltpu.VMEM((CHUNK,), jnp.float32),
            pltpu.VMEM((CHUNK,), jnp.int32),
            pltpu.VMEM((S,), jnp.float32),
            pltpu.VMEM((S,), jnp.int32),
            pltpu.VMEM_SHARED((S,), jnp.float32),
        ],
    )
    def k(res_hbm, idx_hbm, out_hbm, res_v, idx_v, acc_v, iota_v, shared):
        c = lax.axis_index("c")
        s = lax.axis_index("s")
        wid = c * NS + s
        pltpu.sync_copy(res_hbm.at[pl.ds(wid * CHUNK, CHUNK)], res_v)
        pltpu.sync_copy(
            idx_hbm.at[pl.ds(phase * NPP + wid * CHUNK, CHUNK)], idx_v)

        @pl.loop(0, S, step=L)
        def _zero(i):
            acc_v[pl.ds(i, L)] = jnp.zeros((L,), jnp.float32)
            iota_v[pl.ds(i, L)] = lax.iota(jnp.int32, L) + i

        @pl.when(s == 0)
        def _init_shared():
            pltpu.sync_copy(acc_v, shared)

        @pl.loop(0, CHUNK, step=L, unroll=8)
        def _scatter(i):
            plsc.addupdate_scatter(
                acc_v, [idx_v[pl.ds(i, L)]], res_v[pl.ds(i, L)])

        plsc.subcore_barrier()
        pltpu.sync_copy(acc_v, shared.at[iota_v], add=True)
        plsc.subcore_barrier()

        @pl.when(s == 0)
        def _writeout():
            pltpu.sync_copy(shared, out_hbm.at[c])

    return k(res_phase, idx_pad)


def kernel(node_scalar, batch, W1, b1, W2, b2):
    b1r = b1.reshape(1, H)
    w2r = W2.reshape(1, H)
    b2r = b2.reshape(1, 1)
    idx_flat = jnp.concatenate(
        [batch.astype(jnp.int32), jnp.zeros((NPAD - N,), jnp.int32)])
    out = jnp.zeros((S,), jnp.float32)
    for p in range(P):
        res_p = _mlp(node_scalar, W1, b1r, w2r, b2r, p)
        partials = _segsum(res_p, idx_flat, p)
        out = out + partials[0] + partials[1]
    return out


# speedup vs baseline: 1.2075x; 1.0403x over previous
import dataclasses
import functools

import jax
import jax.numpy as jnp
from jax import lax
from jax.experimental import pallas as pl
from jax.experimental.pallas import tpu as pltpu
from jax.experimental.pallas import tpu_sc as plsc

N = 100000
D = 128
H = 64
S = 512

NC = 2
NS = 16
L = 16
NW = NC * NS

P = 2
NPAD = 102400
NPP = NPAD // P
CHUNK = NPP // NW

BLK = 10240
GPP = NPP // BLK


def _mlp_body(x_ref, w1_ref, b1_ref, w2_ref, b2_ref, o_ref, *, phase):
    x = x_ref[...]
    h = jnp.dot(x, w1_ref[...], preferred_element_type=jnp.float32)
    h = h + b1_ref[...]
    h = h * jax.nn.sigmoid(h)
    r = jnp.sum(h * w2_ref[...], axis=1) + b2_ref[0, 0]
    gidx = (phase * NPP + pl.program_id(0) * BLK) + lax.iota(jnp.int32, BLK)
    o_ref[...] = jnp.where(gidx < N, r, 0.0)


def _mlp(x, w1, b1r, w2r, b2r, phase):
    return pl.pallas_call(
        functools.partial(_mlp_body, phase=phase),
        grid=(GPP,),
        in_specs=[
            pl.BlockSpec((BLK, D), lambda i: (i + phase * GPP, 0)),
            pl.BlockSpec((D, H), lambda i: (0, 0)),
            pl.BlockSpec((1, H), lambda i: (0, 0)),
            pl.BlockSpec((1, H), lambda i: (0, 0)),
            pl.BlockSpec((1, 1), lambda i: (0, 0)),
        ],
        out_specs=pl.BlockSpec((BLK,), lambda i: (i,)),
        out_shape=jax.ShapeDtypeStruct((NPP,), jnp.float32),
        compiler_params=pltpu.CompilerParams(
            dimension_semantics=("arbitrary",)),
    )(x, w1, b1r, w2r, b2r)


def _segsum(res_phase, idx_pad, phase):
    mesh = plsc.VectorSubcoreMesh(core_axis_name="c", subcore_axis_name="s")
    cp = pltpu.CompilerParams()
    if "needs_layout_passes" in pltpu.CompilerParams.__dataclass_fields__:
        cp = dataclasses.replace(cp, needs_layout_passes=False)

    @functools.partial(
        pl.kernel,
        compiler_params=cp,
        out_type=jax.ShapeDtypeStruct((NC, S), jnp.float32),
        mesh=mesh,
        scratch_types=[
            pltpu.VMEM((CHUNK,), jnp.float32),
            pltpu.VMEM((CHUNK,), jnp.int32),
            pltpu.VMEM((S,), jnp.float32),
            pltpu.VMEM((S,), jnp.int32),
            pltpu.VMEM_SHARED((S,), jnp.float32),
        ],
    )
    def k(res_hbm, idx_hbm, out_hbm, res_v, idx_v, acc_v, iota_v, shared):
        c = lax.axis_index("c")
        s = lax.axis_index("s")
        wid = c * NS + s
        pltpu.sync_copy(res_hbm.at[pl.ds(wid * CHUNK, CHUNK)], res_v)
        pltpu.sync_copy(
            idx_hbm.at[pl.ds(phase * NPP + wid * CHUNK, CHUNK)], idx_v)

        @pl.loop(0, S, step=L)
        def _zero(i):
            acc_v[pl.ds(i, L)] = jnp.zeros((L,), jnp.float32)
            iota_v[pl.ds(i, L)] = lax.iota(jnp.int32, L) + i

        @pl.when(s == 0)
        def _init_shared():
            pltpu.sync_copy(acc_v, shared)

        @pl.loop(0, CHUNK, step=L, unroll=8)
        def _scatter(i):
            plsc.addupdate_scatter(
                acc_v, [idx_v[pl.ds(i, L)]], res_v[pl.ds(i, L)])

        plsc.subcore_barrier()
        pltpu.sync_copy(acc_v, shared.at[iota_v], add=True)
        plsc.subcore_barrier()

        @pl.when(s == 0)
        def _writeout():
            pltpu.sync_copy(shared, out_hbm.at[c])

    return k(res_phase, idx_pad)


def kernel(node_scalar, batch, W1, b1, W2, b2):
    b1r = b1.reshape(1, H)
    w2r = W2.reshape(1, H)
    b2r = b2.reshape(1, 1)
    idx_flat = jnp.concatenate(
        [batch.astype(jnp.int32), jnp.zeros((NPAD - N,), jnp.int32)])
    out = jnp.zeros((S,), jnp.float32)
    for p in range(P):
        res_p = _mlp(node_scalar, W1, b1r, w2r, b2r, p)
        partials = _segsum(res_p, idx_flat, p)
        out = out + partials[0] + partials[1]
    return out
